# baseline (speedup 1.0000x reference)
import functools

import jax
import jax.numpy as jnp
from jax import lax
from jax.experimental import pallas as pl
from jax.experimental.pallas import tpu as pltpu
from jax.experimental.pallas import tpu_sc as plsc

N = 10000
E = 160000
D = 256

NC = 2
NS = 16
L = 16
NW = NC * NS

NCHUNK = D // L
NR = 320
NPAD = NW * NR
ACC_R = NR + 1
PACK = 16384
BINW = 64
BUFW = BINW + L
BINCAP = 5120
SEPW = 5008
GB2 = 64

EB = 80
EPT = E // NS
NBLK = EPT // EB

_MESH = plsc.VectorSubcoreMesh(core_axis_name="c", subcore_axis_name="s")


@functools.partial(
    pl.kernel,
    out_type=[
        jax.ShapeDtypeStruct((NW * NW * BINCAP,), jnp.int32),
        jax.ShapeDtypeStruct((NW * NW,), jnp.int32),
    ],
    mesh=_MESH,
    scratch_types=[
        pltpu.VMEM((NW, BUFW), jnp.int32),
        pltpu.VMEM((SEPW,), jnp.int32),
        pltpu.VMEM((SEPW,), jnp.int32),
        pltpu.VMEM((NW + L,), jnp.int32),
        pltpu.SMEM((NW,), jnp.int32),
    ],
)
def _bin_edges(bei_hbm, bins_hbm, counts_hbm, buf, src_v, dst_v, cntw, cnts_sm):
    c = lax.axis_index("c")
    s = lax.axis_index("s")
    w = c * NS + s

    ones_i = jnp.ones((L,), jnp.int32)

    def z_cnt(i, _):
        cnts_sm[i] = 0
        return 0

    lax.fori_loop(0, NW, z_cnt, 0)

    nchunk = jnp.where(w < 16, 313, 312)
    base = pl.multiple_of(16 * (w * 312 + jnp.minimum(w, 16)), 16)
    pltpu.sync_copy(bei_hbm.at[pl.ds(base, SEPW)], src_v)
    pltpu.sync_copy(bei_hbm.at[pl.ds(E + base, SEPW)], dst_v)

    def chunk(t, _):
        s16 = src_v[pl.ds(t * L, L)]
        d16 = dst_v[pl.ds(t * L, L)]
        comb = s16 * PACK + d16
        b16 = jnp.right_shift(d16 * 13108, 22)
        for k in range(L):
            bk = b16[k]
            ck = comb[k]
            cnt = cnts_sm[bk]
            buf[bk, pl.ds(cnt & (BINW - 1), L)] = ones_i * ck
            cnts_sm[bk] = cnt + 1

            @pl.when((cnt & (BINW - 1)) == BINW - 1)
            def _():
                off = pl.multiple_of(
                    (w * NW + bk) * BINCAP + (cnt - (BINW - 1)), BINW)
                pltpu.sync_copy(buf.at[bk, pl.ds(0, BINW)],
                                bins_hbm.at[pl.ds(off, BINW)])
        return 0

    lax.fori_loop(0, nchunk, chunk, 0)

    for bk in range(NW):
        cnt = cnts_sm[bk]

        @pl.when((cnt & (BINW - 1)) != 0)
        def _():
            off = pl.multiple_of(
                (w * NW + bk) * BINCAP + (cnt - (cnt & (BINW - 1))), BINW)
            pltpu.sync_copy(buf.at[bk, pl.ds(0, BINW)],
                            bins_hbm.at[pl.ds(off, BINW)])
        cntw[pl.ds(bk, L)] = ones_i * cnt

    pltpu.sync_copy(cntw.at[pl.ds(0, NW)], counts_hbm.at[pl.ds(w * NW, NW)])


@functools.partial(
    pl.kernel,
    out_type=[
        jax.ShapeDtypeStruct((NPAD, D), jnp.float32),
        jax.ShapeDtypeStruct((NPAD,), jnp.float32),
    ],
    mesh=_MESH,
    scratch_types=[
        pltpu.VMEM((GB2, D), jnp.float32),
        pltpu.VMEM((ACC_R, D), jnp.float32),
        pltpu.VMEM((NR + L,), jnp.float32),
        pltpu.SMEM((ACC_R,), jnp.float32),
        pltpu.VMEM((GB2,), jnp.int32),
        pltpu.VMEM((GB2,), jnp.int32),
        pltpu.VMEM((NW * NW + L,), jnp.int32),
        pltpu.SemaphoreType.DMA,
    ],
)
def _seg_accum(x_hbm, bins_hbm, counts_hbm, agg_hbm, deg_hbm,
               rows_v, acc_v, degv, dega_sm, rec_v, sg_v, cnt_v, sem):
    c = lax.axis_index("c")
    s = lax.axis_index("s")
    w = c * NS + s
    lo = w * NR

    zero = jnp.zeros((L,), jnp.float32)
    one = jnp.ones((L,), jnp.float32)
    ones_i = jnp.ones((L,), jnp.int32)
    lane = lax.iota(jnp.int32, L)

    def z_acc(i, _):
        for j in range(NCHUNK):
            acc_v[i, pl.ds(j * L, L)] = zero
        dega_sm[i] = 0.0
        return 0

    lax.fori_loop(0, ACC_R, z_acc, 0)

    def z_deg(i, _):
        degv[pl.ds(i * L, L)] = zero
        return 0

    lax.fori_loop(0, (NR + L) // L, z_deg, 0)

    pltpu.sync_copy(counts_hbm, cnt_v.at[pl.ds(0, NW * NW)])

    def ploop(p, _):
        cntp = cnt_v[pl.ds(p * NW + w, L)][0]
        rbase = (p * NW + w) * BINCAP
        nb = (cntp + (GB2 - 1)) // GB2

        def bblk(bb, _):
            pltpu.sync_copy(bins_hbm.at[pl.ds(rbase + bb * GB2, GB2)], rec_v)
            for t2 in range(GB2 // L):
                rc = rec_v[pl.ds(t2 * L, L)]
                idxv = lane + (bb * GB2 + t2 * L)
                okm = idxv < (ones_i * cntp)
                sg_v[pl.ds(t2 * L, L)] = jnp.where(
                    okm, jnp.right_shift(rc, 14), 0)
            pltpu.async_copy(x_hbm.at[sg_v], rows_v, sem).wait()

            def grp(q, _):
                gbase = bb * GB2 + q * L
                rc = rec_v[pl.ds(q * L, L)]
                d16 = jnp.bitwise_and(rc, PACK - 1)
                for k in range(L):
                    e = gbase + k
                    row = jnp.where(e < cntp, d16[k] - lo, NR)
                    er = q * L + k
                    for j in range(NCHUNK):
                        sl = pl.ds(j * L, L)
                        acc_v[row, sl] = acc_v[row, sl] + rows_v[er, sl]
                    dega_sm[row] = dega_sm[row] + 1.0
                return 0

            lax.fori_loop(0, GB2 // L, grp, 0)
            return 0

        lax.fori_loop(0, nb, bblk, 0)
        return 0

    lax.fori_loop(0, NW, ploop, 0)

    def fin(i, _):
        degv[pl.ds(i, L)] = one * dega_sm[i]
        return 0

    lax.fori_loop(0, NR, fin, 0)
    pltpu.sync_copy(acc_v.at[pl.ds(0, NR)], agg_hbm.at[pl.ds(w * NR, NR)])
    pltpu.sync_copy(degv.at[pl.ds(0, NR)], deg_hbm.at[pl.ds(w * NR, NR)])


def _emb_body(x_ref, agg_ref, deg_ref, ws_ref, wn_ref, h_ref):
    deg = deg_ref[...]
    scale = 1.0 / jnp.maximum(deg, 1.0)
    a = agg_ref[...] * scale
    h = jnp.dot(x_ref[...], ws_ref[...], preferred_element_type=jnp.float32)
    h = h + jnp.dot(a, wn_ref[...], preferred_element_type=jnp.float32)
    h_ref[...] = jnp.maximum(h, 0.0)


_ROWS_BLK = 1000


def _emb(x, agg, degw, W_self, W_neigh):
    return pl.pallas_call(
        _emb_body,
        grid=(N // _ROWS_BLK,),
        in_specs=[
            pl.BlockSpec((_ROWS_BLK, D), lambda i: (i, 0)),
            pl.BlockSpec((_ROWS_BLK, D), lambda i: (i, 0)),
            pl.BlockSpec((_ROWS_BLK, 1), lambda i: (i, 0)),
            pl.BlockSpec((D, D), lambda i: (0, 0)),
            pl.BlockSpec((D, D), lambda i: (0, 0)),
        ],
        out_specs=pl.BlockSpec((_ROWS_BLK, D), lambda i: (i, 0)),
        out_shape=jax.ShapeDtypeStruct((N, D), jnp.float32),
    )(x, agg, degw, W_self, W_neigh)


@functools.partial(
    pl.kernel,
    out_type=[
        jax.ShapeDtypeStruct((E,), jnp.float32),
        jax.ShapeDtypeStruct((E,), jnp.float32),
    ],
    mesh=_MESH,
    scratch_types=[
        pltpu.VMEM((EB, D), jnp.float32),
        pltpu.VMEM((EB, D), jnp.float32),
        pltpu.VMEM((EB,), jnp.int32),
        pltpu.VMEM((EB,), jnp.int32),
        pltpu.VMEM((D,), jnp.float32),
        pltpu.VMEM((EPT,), jnp.float32),
        pltpu.SemaphoreType.DMA,
        pltpu.SemaphoreType.DMA,
    ],
)
def _score(h_hbm, pos_hbm, neg_hbm, r_hbm, pos_out, neg_out,
           urows, vrows, uidx, vidx, r_v, sbuf, sem_u, sem_v):
    c = lax.axis_index("c")
    s = lax.axis_index("s")

    pltpu.sync_copy(r_hbm, r_v)
    r_regs = [r_v[pl.ds(j * L, L)] for j in range(NCHUNK)]
    lane = lax.iota(jnp.int32, L)
    onehots = [jnp.where(lane == k, 1.0, 0.0) for k in range(L)]

    def do_set(ei_hbm, out_hbm):
        def blk(g, _):
            base = s * EPT + g * EB
            pltpu.sync_copy(ei_hbm.at[pl.ds(base, EB)], uidx)
            pltpu.sync_copy(ei_hbm.at[pl.ds(E + base, EB)], vidx)
            cu = pltpu.async_copy(h_hbm.at[uidx], urows, sem_u)
            cv = pltpu.async_copy(h_hbm.at[vidx], vrows, sem_v)
            cu.wait()
            cv.wait()

            def grp(q, _):
                vec = jnp.zeros((L,), jnp.float32)
                for k in range(L):
                    e = q * L + k
                    sl = pl.ds(0, L)
                    acc = urows[e, sl] * vrows[e, sl] * r_regs[0]
                    for j in range(1, NCHUNK):
                        sl = pl.ds(j * L, L)
                        acc = acc + urows[e, sl] * vrows[e, sl] * r_regs[j]
                    ssum = acc[0]
                    for i in range(1, L):
                        ssum = ssum + acc[i]
                    vec = vec + ssum * onehots[k]
                sbuf[pl.ds(g * EB + q * L, L)] = vec
                return 0

            lax.fori_loop(0, EB // L, grp, 0)
            return 0

        lax.fori_loop(0, NBLK, blk, 0)
        pltpu.sync_copy(sbuf, out_hbm.at[pl.ds(s * EPT, EPT)])

    @pl.when(c == 0)
    def _():
        do_set(pos_hbm, pos_out)

    @pl.when(c == 1)
    def _():
        do_set(neg_hbm, neg_out)


def kernel(x, block_edge_index, pos_edge_index, neg_edge_index, W_self, W_neigh, r):
    bei_flat = jnp.concatenate(
        [block_edge_index.reshape(-1), jnp.zeros((L,), jnp.int32)])
    bins, counts = _bin_edges(bei_flat)
    agg, degw = _seg_accum(x, bins, counts)
    h = _emb(x, agg[:N], degw[:N].reshape(N, 1), W_self, W_neigh)
    pos_score, neg_score = _score(h, pos_edge_index.reshape(-1),
                                  neg_edge_index.reshape(-1), r)
    return (pos_score, neg_score)


# speedup vs baseline: 1.3421x; 1.3421x over previous
import functools

import jax
import jax.numpy as jnp
from jax import lax
from jax.experimental import pallas as pl
from jax.experimental.pallas import tpu as pltpu
from jax.experimental.pallas import tpu_sc as plsc

N = 10000
E = 160000
D = 256

NC = 2
NS = 16
L = 16
NW = NC * NS

NCHUNK = D // L
NR = 320
NPAD = NW * NR
ACC_R = NR + 1
SCB = 2000
NSB = E // SCB
CAP = SCB + L
GB = 32
PACK = 16384

EB = 80
EPT = E // NS
NBLK = EPT // EB

_MESH = plsc.VectorSubcoreMesh(core_axis_name="c", subcore_axis_name="s")


@functools.partial(
    pl.kernel,
    out_type=[
        jax.ShapeDtypeStruct((NPAD, D), jnp.float32),
        jax.ShapeDtypeStruct((NPAD,), jnp.float32),
    ],
    mesh=_MESH,
    scratch_types=[
        pltpu.VMEM((GB, D), jnp.float32),
        pltpu.VMEM((ACC_R, D), jnp.float32),
        pltpu.VMEM((NR + L,), jnp.float32),
        pltpu.SMEM((ACC_R,), jnp.float32),
        pltpu.VMEM((CAP,), jnp.int32),
        pltpu.VMEM((GB,), jnp.int32),
        pltpu.VMEM((SCB,), jnp.int32),
        pltpu.VMEM((SCB,), jnp.int32),
        pltpu.SemaphoreType.DMA,
    ],
)
def _seg_sum(x_hbm, bei_hbm, agg_hbm, deg_hbm,
             rows_v, acc_v, degv, dega_sm, idxc, sg_v, src_v, dst_v, sem):
    c = lax.axis_index("c")
    s = lax.axis_index("s")
    w = c * NS + s
    lo = w * NR

    zero = jnp.zeros((L,), jnp.float32)
    zero_i = jnp.zeros((L,), jnp.int32)
    one = jnp.ones((L,), jnp.float32)
    ones_i = jnp.ones((L,), jnp.int32)

    def z_acc(i, _):
        for j in range(NCHUNK):
            acc_v[i, pl.ds(j * L, L)] = zero
        dega_sm[i] = 0.0
        return 0

    lax.fori_loop(0, ACC_R, z_acc, 0)

    def z_deg(i, _):
        degv[pl.ds(i * L, L)] = zero
        return 0

    lax.fori_loop(0, (NR + L) // L, z_deg, 0)

    def z_idx(i, _):
        idxc[pl.ds(i * L, L)] = zero_i
        return 0

    lax.fori_loop(0, CAP // L, z_idx, 0)
    for k2 in range(GB // L):
        sg_v[pl.ds(k2 * L, L)] = zero_i

    def sblk(b, _):
        ebase = b * SCB
        pltpu.sync_copy(bei_hbm.at[pl.ds(ebase, SCB)], src_v)
        pltpu.sync_copy(bei_hbm.at[pl.ds(E + ebase, SCB)], dst_v)

        def chunk(t, cnt):
            s16 = src_v[pl.ds(t * L, L)]
            d16 = dst_v[pl.ds(t * L, L)]
            comb = s16 * PACK + d16
            okv = (d16 >= lo) & (d16 < lo + NR)
            oki = jnp.where(okv, 1, 0)
            for k in range(L):
                idxc[pl.ds(cnt, L)] = ones_i * comb[k]
                cnt = cnt + oki[k]
            return cnt

        cnt = lax.fori_loop(0, SCB // L, chunk, jnp.int32(0))

        nb = (cnt + (GB - 1)) // GB

        def gblk(bb, _):
            for k2 in range(GB // L):
                cb0 = idxc[pl.ds(bb * GB + k2 * L, L)]
                sg_v[pl.ds(k2 * L, L)] = jnp.right_shift(cb0, 14)
            pltpu.async_copy(x_hbm.at[sg_v], rows_v, sem).wait()

            def grp(q, _):
                gbase = bb * GB + q * L
                cb = idxc[pl.ds(gbase, L)]
                d16 = jnp.bitwise_and(cb, PACK - 1)
                for k in range(L):
                    e = gbase + k
                    row = jnp.where(e < cnt, d16[k] - lo, NR)
                    er = q * L + k
                    for j in range(NCHUNK):
                        sl = pl.ds(j * L, L)
                        acc_v[row, sl] = acc_v[row, sl] + rows_v[er, sl]
                    dega_sm[row] = dega_sm[row] + 1.0
                return 0

            lax.fori_loop(0, GB // L, grp, 0)
            return 0

        lax.fori_loop(0, nb, gblk, 0)
        return 0

    lax.fori_loop(0, NSB, sblk, 0)

    def fin(i, _):
        degv[pl.ds(i, L)] = one * dega_sm[i]
        return 0

    lax.fori_loop(0, NR, fin, 0)
    pltpu.sync_copy(acc_v.at[pl.ds(0, NR)], agg_hbm.at[pl.ds(w * NR, NR)])
    pltpu.sync_copy(degv.at[pl.ds(0, NR)], deg_hbm.at[pl.ds(w * NR, NR)])


def _emb_body(x_ref, agg_ref, deg_ref, ws_ref, wn_ref, h_ref):
    deg = deg_ref[...]
    scale = 1.0 / jnp.maximum(deg, 1.0)
    a = agg_ref[...] * scale
    h = jnp.dot(x_ref[...], ws_ref[...], preferred_element_type=jnp.float32)
    h = h + jnp.dot(a, wn_ref[...], preferred_element_type=jnp.float32)
    h_ref[...] = jnp.maximum(h, 0.0).astype(jnp.bfloat16)


_ROWS_BLK = 1000


def _emb(x, agg, degw, W_self, W_neigh):
    return pl.pallas_call(
        _emb_body,
        grid=(N // _ROWS_BLK,),
        in_specs=[
            pl.BlockSpec((_ROWS_BLK, D), lambda i: (i, 0)),
            pl.BlockSpec((_ROWS_BLK, D), lambda i: (i, 0)),
            pl.BlockSpec((_ROWS_BLK, 1), lambda i: (i, 0)),
            pl.BlockSpec((D, D), lambda i: (0, 0)),
            pl.BlockSpec((D, D), lambda i: (0, 0)),
        ],
        out_specs=pl.BlockSpec((_ROWS_BLK, D), lambda i: (i, 0)),
        out_shape=jax.ShapeDtypeStruct((N, D), jnp.bfloat16),
    )(x, agg, degw, W_self, W_neigh)


DH = D // 2


@functools.partial(
    pl.kernel,
    out_type=[
        jax.ShapeDtypeStruct((E,), jnp.float32),
        jax.ShapeDtypeStruct((E,), jnp.float32),
    ],
    mesh=_MESH,
    scratch_types=[
        pltpu.VMEM((EB, DH), jnp.int32),
        pltpu.VMEM((EB, DH), jnp.int32),
        pltpu.VMEM((EB, DH), jnp.int32),
        pltpu.VMEM((EB, DH), jnp.int32),
        pltpu.VMEM((EB,), jnp.int32),
        pltpu.VMEM((EB,), jnp.int32),
        pltpu.VMEM((EB,), jnp.int32),
        pltpu.VMEM((EB,), jnp.int32),
        pltpu.VMEM((DH,), jnp.float32),
        pltpu.VMEM((DH,), jnp.float32),
        pltpu.VMEM((EPT,), jnp.float32),
        pltpu.SemaphoreType.DMA,
        pltpu.SemaphoreType.DMA,
        pltpu.SemaphoreType.DMA,
        pltpu.SemaphoreType.DMA,
    ],
)
def _score(h_hbm, pos_hbm, neg_hbm, re_hbm, ro_hbm, pos_out, neg_out,
           ua, va, ub, vb, uia, via, uib, vib, re_v, ro_v, sbuf,
           sua, sva, sub_, svb):
    c = lax.axis_index("c")
    s = lax.axis_index("s")

    pltpu.sync_copy(re_hbm, re_v)
    pltpu.sync_copy(ro_hbm, ro_v)
    re_regs = [re_v[pl.ds(j * L, L)] for j in range(DH // L)]
    ro_regs = [ro_v[pl.ds(j * L, L)] for j in range(DH // L)]
    lane = lax.iota(jnp.int32, L)
    onehots = [jnp.where(lane == k, 1.0, 0.0) for k in range(L)]

    def do_set(ei_hbm, out_hbm):
        def fire(b, ui, vi, ur, vr, su, sv):
            base = s * EPT + b * EB
            pltpu.sync_copy(ei_hbm.at[pl.ds(base, EB)], ui)
            pltpu.sync_copy(ei_hbm.at[pl.ds(E + base, EB)], vi)
            cu = pltpu.async_copy(h_hbm.at[ui], ur, su)
            cv = pltpu.async_copy(h_hbm.at[vi], vr, sv)
            return cu, cv

        def compute(b, ur, vr):
            def grp(q, _):
                vec = jnp.zeros((L,), jnp.float32)
                for k in range(L):
                    e = q * L + k
                    acc = None
                    for j in range(DH // L):
                        sl = pl.ds(j * L, L)
                        uw = ur[e, sl]
                        vw = vr[e, sl]
                        ulo = jax.lax.bitcast_convert_type(
                            jax.lax.shift_left(uw, 16), jnp.float32)
                        uhi = jax.lax.bitcast_convert_type(uw, jnp.float32)
                        vlo = jax.lax.bitcast_convert_type(
                            jax.lax.shift_left(vw, 16), jnp.float32)
                        vhi = jax.lax.bitcast_convert_type(vw, jnp.float32)
                        t = ulo * vlo * re_regs[j] + uhi * vhi * ro_regs[j]
                        acc = t if acc is None else acc + t
                    ssum = acc[0]
                    for i in range(1, L):
                        ssum = ssum + acc[i]
                    vec = vec + ssum * onehots[k]
                sbuf[pl.ds(b * EB + q * L, L)] = vec
                return 0

            lax.fori_loop(0, EB // L, grp, 0)

        ca = fire(0, uia, via, ua, va, sua, sva)

        def wait(ur, su):
            pltpu.make_async_copy(h_hbm.at[uia], ur, su).wait()

        def pair(i, _):
            b = 2 * i
            wait(ua, sua)
            wait(va, sva)
            fire(b + 1, uib, vib, ub, vb, sub_, svb)
            compute(b, ua, va)
            wait(ub, sub_)
            wait(vb, svb)
            fire(b + 2, uia, via, ua, va, sua, sva)
            compute(b + 1, ub, vb)
            return 0

        lax.fori_loop(0, (NBLK - 1) // 2, pair, 0)
        wait(ua, sua)
        wait(va, sva)
        compute(NBLK - 1, ua, va)
        pltpu.sync_copy(sbuf, out_hbm.at[pl.ds(s * EPT, EPT)])

    @pl.when(c == 0)
    def _():
        do_set(pos_hbm, pos_out)

    @pl.when(c == 1)
    def _():
        do_set(neg_hbm, neg_out)


def kernel(x, block_edge_index, pos_edge_index, neg_edge_index, W_self, W_neigh, r):
    agg, degw = _seg_sum(x, block_edge_index.reshape(-1))
    h = _emb(x, agg[:N], degw[:N].reshape(N, 1), W_self, W_neigh)
    h32 = jax.lax.bitcast_convert_type(h.reshape(N, DH, 2), jnp.int32)
    pos_score, neg_score = _score(h32, pos_edge_index.reshape(-1),
                                  neg_edge_index.reshape(-1),
                                  r[0::2], r[1::2])
    return (pos_score, neg_score)


# speedup vs baseline: 1.3431x; 1.0008x over previous
import functools

import jax
import jax.numpy as jnp
from jax import lax
from jax.experimental import pallas as pl
from jax.experimental.pallas import tpu as pltpu
from jax.experimental.pallas import tpu_sc as plsc

N = 10000
E = 160000
D = 256

NC = 2
NS = 16
L = 16
NW = NC * NS

NCHUNK = D // L
NR = 320
NPAD = NW * NR
ACC_R = NR + 1
SCB = 2000
NSB = E // SCB
CAP = SCB + L
GB = 32
PACK = 16384

EB = 80
EPT = E // NS
NBLK = EPT // EB

_MESH = plsc.VectorSubcoreMesh(core_axis_name="c", subcore_axis_name="s")


@functools.partial(
    pl.kernel,
    out_type=[
        jax.ShapeDtypeStruct((NPAD, D), jnp.float32),
        jax.ShapeDtypeStruct((NPAD,), jnp.float32),
    ],
    mesh=_MESH,
    scratch_types=[
        pltpu.VMEM((GB, D), jnp.float32),
        pltpu.VMEM((ACC_R, D), jnp.float32),
        pltpu.VMEM((NR + L,), jnp.float32),
        pltpu.SMEM((ACC_R,), jnp.float32),
        pltpu.VMEM((CAP,), jnp.int32),
        pltpu.VMEM((GB,), jnp.int32),
        pltpu.VMEM((SCB,), jnp.int32),
        pltpu.VMEM((SCB,), jnp.int32),
        pltpu.SemaphoreType.DMA,
    ],
)
def _seg_sum(x_hbm, bei_hbm, agg_hbm, deg_hbm,
             rows_v, acc_v, degv, dega_sm, idxc, sg_v, src_v, dst_v, sem):
    c = lax.axis_index("c")
    s = lax.axis_index("s")
    w = c * NS + s
    lo = w * NR

    zero = jnp.zeros((L,), jnp.float32)
    zero_i = jnp.zeros((L,), jnp.int32)
    one = jnp.ones((L,), jnp.float32)
    ones_i = jnp.ones((L,), jnp.int32)

    def z_acc(i, _):
        for j in range(NCHUNK):
            acc_v[i, pl.ds(j * L, L)] = zero
        dega_sm[i] = 0.0
        return 0

    lax.fori_loop(0, ACC_R, z_acc, 0)

    def z_deg(i, _):
        degv[pl.ds(i * L, L)] = zero
        return 0

    lax.fori_loop(0, (NR + L) // L, z_deg, 0)

    def z_idx(i, _):
        idxc[pl.ds(i * L, L)] = zero_i
        return 0

    lax.fori_loop(0, CAP // L, z_idx, 0)
    for k2 in range(GB // L):
        sg_v[pl.ds(k2 * L, L)] = zero_i

    def sblk(b, _):
        ebase = b * SCB
        pltpu.sync_copy(bei_hbm.at[pl.ds(ebase, SCB)], src_v)
        pltpu.sync_copy(bei_hbm.at[pl.ds(E + ebase, SCB)], dst_v)

        def chunk(t, cnt):
            s16 = src_v[pl.ds(t * L, L)]
            d16 = dst_v[pl.ds(t * L, L)]
            comb = s16 * PACK + d16
            okv = (d16 >= lo) & (d16 < lo + NR)
            oki = jnp.where(okv, 1, 0)
            for k in range(L):
                idxc[pl.ds(cnt, L)] = ones_i * comb[k]
                cnt = cnt + oki[k]
            return cnt

        cnt = lax.fori_loop(0, SCB // L, chunk, jnp.int32(0))

        nb = (cnt + (GB - 1)) // GB

        def gblk(bb, _):
            for k2 in range(GB // L):
                cb0 = idxc[pl.ds(bb * GB + k2 * L, L)]
                sg_v[pl.ds(k2 * L, L)] = jnp.right_shift(cb0, 14)
            pltpu.async_copy(x_hbm.at[sg_v], rows_v, sem).wait()

            def grp(q, _):
                gbase = bb * GB + q * L
                cb = idxc[pl.ds(gbase, L)]
                d16 = jnp.bitwise_and(cb, PACK - 1)
                for k in range(L):
                    e = gbase + k
                    row = jnp.where(e < cnt, d16[k] - lo, NR)
                    er = q * L + k
                    for j in range(NCHUNK):
                        sl = pl.ds(j * L, L)
                        acc_v[row, sl] = acc_v[row, sl] + rows_v[er, sl]
                    dega_sm[row] = dega_sm[row] + 1.0
                return 0

            lax.fori_loop(0, GB // L, grp, 0)
            return 0

        lax.fori_loop(0, nb, gblk, 0)
        return 0

    lax.fori_loop(0, NSB, sblk, 0)

    def fin(i, _):
        degv[pl.ds(i, L)] = one * dega_sm[i]
        return 0

    lax.fori_loop(0, NR, fin, 0)
    pltpu.sync_copy(acc_v.at[pl.ds(0, NR)], agg_hbm.at[pl.ds(w * NR, NR)])
    pltpu.sync_copy(degv.at[pl.ds(0, NR)], deg_hbm.at[pl.ds(w * NR, NR)])


def _emb_body(x_ref, agg_ref, deg_ref, ws_ref, wn_ref, h_ref):
    deg = deg_ref[...]
    scale = 1.0 / jnp.maximum(deg, 1.0)
    a = agg_ref[...] * scale
    h = jnp.dot(x_ref[...], ws_ref[...], preferred_element_type=jnp.float32)
    h = h + jnp.dot(a, wn_ref[...], preferred_element_type=jnp.float32)
    h_ref[...] = jnp.maximum(h, 0.0).astype(jnp.bfloat16)


_ROWS_BLK = 1000


def _emb(x, agg, degw, W_self, W_neigh):
    return pl.pallas_call(
        _emb_body,
        grid=(N // _ROWS_BLK,),
        in_specs=[
            pl.BlockSpec((_ROWS_BLK, D), lambda i: (i, 0)),
            pl.BlockSpec((_ROWS_BLK, D), lambda i: (i, 0)),
            pl.BlockSpec((_ROWS_BLK, 1), lambda i: (i, 0)),
            pl.BlockSpec((D, D), lambda i: (0, 0)),
            pl.BlockSpec((D, D), lambda i: (0, 0)),
        ],
        out_specs=pl.BlockSpec((_ROWS_BLK, D), lambda i: (i, 0)),
        out_shape=jax.ShapeDtypeStruct((N, D), jnp.bfloat16),
    )(x, agg, degw, W_self, W_neigh)


DH = D // 2


@functools.partial(
    pl.kernel,
    out_type=[
        jax.ShapeDtypeStruct((E,), jnp.float32),
        jax.ShapeDtypeStruct((E,), jnp.float32),
    ],
    mesh=_MESH,
    scratch_types=[
        pltpu.VMEM((EB, DH), jnp.int32),
        pltpu.VMEM((EB, DH), jnp.int32),
        pltpu.VMEM((EB, DH), jnp.int32),
        pltpu.VMEM((EB, DH), jnp.int32),
        pltpu.VMEM((EB,), jnp.int32),
        pltpu.VMEM((EB,), jnp.int32),
        pltpu.VMEM((EB,), jnp.int32),
        pltpu.VMEM((EB,), jnp.int32),
        pltpu.VMEM((DH,), jnp.float32),
        pltpu.VMEM((DH,), jnp.float32),
        pltpu.VMEM((EPT,), jnp.float32),
        pltpu.SemaphoreType.DMA,
        pltpu.SemaphoreType.DMA,
        pltpu.SemaphoreType.DMA,
        pltpu.SemaphoreType.DMA,
    ],
)
def _score(h_hbm, pos_hbm, neg_hbm, re_hbm, ro_hbm, pos_out, neg_out,
           ua, va, ub, vb, uia, via, uib, vib, re_v, ro_v, sbuf,
           sua, sva, sub_, svb):
    c = lax.axis_index("c")
    s = lax.axis_index("s")

    pltpu.sync_copy(re_hbm, re_v)
    pltpu.sync_copy(ro_hbm, ro_v)
    re_regs = [re_v[pl.ds(j * L, L)] for j in range(DH // L)]
    ro_regs = [ro_v[pl.ds(j * L, L)] for j in range(DH // L)]
    lane = lax.iota(jnp.int32, L)
    onehots = [jnp.where(lane == k, 1.0, 0.0) for k in range(L)]

    def do_set(ei_hbm, out_hbm):
        def fire(b, ui, vi, ur, vr, su, sv):
            base = s * EPT + b * EB
            pltpu.sync_copy(ei_hbm.at[pl.ds(base, EB)], ui)
            pltpu.sync_copy(ei_hbm.at[pl.ds(E + base, EB)], vi)
            cu = pltpu.async_copy(h_hbm.at[ui], ur, su)
            cv = pltpu.async_copy(h_hbm.at[vi], vr, sv)
            return cu, cv

        def compute(b, ur, vr):
            def grp(q, _):
                vec = jnp.zeros((L,), jnp.float32)
                for k in range(L):
                    e = q * L + k
                    acc = None
                    for j in range(DH // L):
                        sl = pl.ds(j * L, L)
                        uw = ur[e, sl]
                        vw = vr[e, sl]
                        ulo = jax.lax.bitcast_convert_type(
                            jax.lax.shift_left(uw, 16), jnp.float32)
                        uhi = jax.lax.bitcast_convert_type(uw, jnp.float32)
                        vlo = jax.lax.bitcast_convert_type(
                            jax.lax.shift_left(vw, 16), jnp.float32)
                        vhi = jax.lax.bitcast_convert_type(vw, jnp.float32)
                        t = ulo * vlo * re_regs[j] + uhi * vhi * ro_regs[j]
                        acc = t if acc is None else acc + t
                    lanes = [acc[i] for i in range(L)]
                    while len(lanes) > 1:
                        lanes = [lanes[i] + lanes[i + 1]
                                 for i in range(0, len(lanes), 2)]
                    vec = vec + lanes[0] * onehots[k]
                sbuf[pl.ds(b * EB + q * L, L)] = vec
                return 0

            lax.fori_loop(0, EB // L, grp, 0)

        ca = fire(0, uia, via, ua, va, sua, sva)

        def wait(ur, su):
            pltpu.make_async_copy(h_hbm.at[uia], ur, su).wait()

        def pair(i, _):
            b = 2 * i
            wait(ua, sua)
            wait(va, sva)
            fire(b + 1, uib, vib, ub, vb, sub_, svb)
            compute(b, ua, va)
            wait(ub, sub_)
            wait(vb, svb)
            fire(b + 2, uia, via, ua, va, sua, sva)
            compute(b + 1, ub, vb)
            return 0

        lax.fori_loop(0, (NBLK - 1) // 2, pair, 0)
        wait(ua, sua)
        wait(va, sva)
        compute(NBLK - 1, ua, va)
        pltpu.sync_copy(sbuf, out_hbm.at[pl.ds(s * EPT, EPT)])

    @pl.when(c == 0)
    def _():
        do_set(pos_hbm, pos_out)

    @pl.when(c == 1)
    def _():
        do_set(neg_hbm, neg_out)


def kernel(x, block_edge_index, pos_edge_index, neg_edge_index, W_self, W_neigh, r):
    agg, degw = _seg_sum(x, block_edge_index.reshape(-1))
    h = _emb(x, agg[:N], degw[:N].reshape(N, 1), W_self, W_neigh)
    h32 = jax.lax.bitcast_convert_type(h.reshape(N, DH, 2), jnp.int32)
    pos_score, neg_score = _score(h32, pos_edge_index.reshape(-1),
                                  neg_edge_index.reshape(-1),
                                  r[0::2], r[1::2])
    return (pos_score, neg_score)


# speedup vs baseline: 1.4486x; 1.0785x over previous
import functools

import jax
import jax.numpy as jnp
from jax import lax
from jax.experimental import pallas as pl
from jax.experimental.pallas import tpu as pltpu
from jax.experimental.pallas import tpu_sc as plsc

N = 10000
E = 160000
D = 256

NC = 2
NS = 16
L = 16
NW = NC * NS

NCHUNK = D // L
NR = 320
NPAD = NW * NR
ACC_R = NR + 1
SCB = 2000
NSB = E // SCB
CAP = SCB + L
GB = 80
PACK = 16384

EB = 80
EPT = E // NS
NBLK = EPT // EB

_MESH = plsc.VectorSubcoreMesh(core_axis_name="c", subcore_axis_name="s")


@functools.partial(
    pl.kernel,
    out_type=[
        jax.ShapeDtypeStruct((NPAD, D), jnp.float32),
        jax.ShapeDtypeStruct((NPAD,), jnp.float32),
    ],
    mesh=_MESH,
    scratch_types=[
        pltpu.VMEM((GB, D // 2), jnp.int32),
        pltpu.VMEM((ACC_R, D), jnp.float32),
        pltpu.VMEM((NR + L,), jnp.float32),
        pltpu.SMEM((ACC_R,), jnp.float32),
        pltpu.VMEM((CAP,), jnp.int32),
        pltpu.VMEM((GB,), jnp.int32),
        pltpu.VMEM((SCB,), jnp.int32),
        pltpu.VMEM((SCB,), jnp.int32),
        pltpu.SemaphoreType.DMA,
    ],
)
def _seg_sum(x_hbm, bei_hbm, agg_hbm, deg_hbm,
             rows_v, acc_v, degv, dega_sm, idxc, sg_v, src_v, dst_v, sem):
    c = lax.axis_index("c")
    s = lax.axis_index("s")
    w = c * NS + s
    lo = w * NR

    zero = jnp.zeros((L,), jnp.float32)
    zero_i = jnp.zeros((L,), jnp.int32)
    one = jnp.ones((L,), jnp.float32)
    ones_i = jnp.ones((L,), jnp.int32)

    def z_acc(i, _):
        for j in range(NCHUNK):
            acc_v[i, pl.ds(j * L, L)] = zero
        dega_sm[i] = 0.0
        return 0

    lax.fori_loop(0, ACC_R, z_acc, 0)

    def z_deg(i, _):
        degv[pl.ds(i * L, L)] = zero
        return 0

    lax.fori_loop(0, (NR + L) // L, z_deg, 0)

    def z_idx(i, _):
        idxc[pl.ds(i * L, L)] = zero_i
        return 0

    lax.fori_loop(0, CAP // L, z_idx, 0)
    for k2 in range(GB // L):
        sg_v[pl.ds(k2 * L, L)] = zero_i

    def sblk(b, _):
        ebase = b * SCB
        pltpu.sync_copy(bei_hbm.at[pl.ds(ebase, SCB)], src_v)
        pltpu.sync_copy(bei_hbm.at[pl.ds(E + ebase, SCB)], dst_v)

        def chunk(t, cnt):
            s16 = src_v[pl.ds(t * L, L)]
            d16 = dst_v[pl.ds(t * L, L)]
            comb = s16 * PACK + d16
            okv = (d16 >= lo) & (d16 < lo + NR)
            oki = jnp.where(okv, 1, 0)
            for k in range(L):
                idxc[pl.ds(cnt, L)] = ones_i * comb[k]
                cnt = cnt + oki[k]
            return cnt

        cnt = lax.fori_loop(0, SCB // L, chunk, jnp.int32(0))

        nb = (cnt + (GB - 1)) // GB

        def gblk(bb, _):
            for k2 in range(GB // L):
                cb0 = idxc[pl.ds(bb * GB + k2 * L, L)]
                sg_v[pl.ds(k2 * L, L)] = jnp.right_shift(cb0, 14)
            pltpu.async_copy(x_hbm.at[sg_v], rows_v, sem).wait()

            def grp(q, _):
                gbase = bb * GB + q * L
                cb = idxc[pl.ds(gbase, L)]
                d16 = jnp.bitwise_and(cb, PACK - 1)
                for k in range(L):
                    e = gbase + k
                    row = jnp.where(e < cnt, d16[k] - lo, NR)
                    er = q * L + k
                    for j in range(NCHUNK // 2):
                        sl = pl.ds(j * L, L)
                        w32 = rows_v[er, sl]
                        wlo = jax.lax.bitcast_convert_type(
                            jax.lax.shift_left(w32, 16), jnp.float32)
                        whi = jax.lax.bitcast_convert_type(
                            jnp.bitwise_and(w32, -65536), jnp.float32)
                        slh = pl.ds((NCHUNK // 2 + j) * L, L)
                        acc_v[row, sl] = acc_v[row, sl] + wlo
                        acc_v[row, slh] = acc_v[row, slh] + whi
                    dega_sm[row] = dega_sm[row] + 1.0
                return 0

            lax.fori_loop(0, GB // L, grp, 0)
            return 0

        lax.fori_loop(0, nb, gblk, 0)
        return 0

    lax.fori_loop(0, NSB, sblk, 0)

    def fin(i, _):
        degv[pl.ds(i, L)] = one * dega_sm[i]
        return 0

    lax.fori_loop(0, NR, fin, 0)
    pltpu.sync_copy(acc_v.at[pl.ds(0, NR)], agg_hbm.at[pl.ds(w * NR, NR)])
    pltpu.sync_copy(degv.at[pl.ds(0, NR)], deg_hbm.at[pl.ds(w * NR, NR)])


def _emb_body(x_ref, agg_ref, deg_ref, ws_ref, wn_ref, h_ref):
    deg = deg_ref[...]
    scale = 1.0 / jnp.maximum(deg, 1.0)
    a = agg_ref[...] * scale
    h = jnp.dot(x_ref[...], ws_ref[...], preferred_element_type=jnp.float32)
    h = h + jnp.dot(a, wn_ref[...], preferred_element_type=jnp.float32)
    h_ref[...] = jnp.maximum(h, 0.0).astype(jnp.bfloat16)


_ROWS_BLK = 1000


def _emb(x, agg, degw, W_self, W_neigh):
    return pl.pallas_call(
        _emb_body,
        grid=(N // _ROWS_BLK,),
        in_specs=[
            pl.BlockSpec((_ROWS_BLK, D), lambda i: (i, 0)),
            pl.BlockSpec((_ROWS_BLK, D), lambda i: (i, 0)),
            pl.BlockSpec((_ROWS_BLK, 1), lambda i: (i, 0)),
            pl.BlockSpec((D, D), lambda i: (0, 0)),
            pl.BlockSpec((D, D), lambda i: (0, 0)),
        ],
        out_specs=pl.BlockSpec((_ROWS_BLK, D), lambda i: (i, 0)),
        out_shape=jax.ShapeDtypeStruct((N, D), jnp.bfloat16),
    )(x, agg, degw, W_self, W_neigh)


DH = D // 2


@functools.partial(
    pl.kernel,
    out_type=[
        jax.ShapeDtypeStruct((E,), jnp.float32),
        jax.ShapeDtypeStruct((E,), jnp.float32),
    ],
    mesh=_MESH,
    scratch_types=[
        pltpu.VMEM((EB, DH), jnp.int32),
        pltpu.VMEM((EB, DH), jnp.int32),
        pltpu.VMEM((EB, DH), jnp.int32),
        pltpu.VMEM((EB, DH), jnp.int32),
        pltpu.VMEM((EB,), jnp.int32),
        pltpu.VMEM((EB,), jnp.int32),
        pltpu.VMEM((EB,), jnp.int32),
        pltpu.VMEM((EB,), jnp.int32),
        pltpu.VMEM((DH,), jnp.float32),
        pltpu.VMEM((DH,), jnp.float32),
        pltpu.VMEM((EPT,), jnp.float32),
        pltpu.SemaphoreType.DMA,
        pltpu.SemaphoreType.DMA,
        pltpu.SemaphoreType.DMA,
        pltpu.SemaphoreType.DMA,
    ],
)
def _score(h_hbm, pos_hbm, neg_hbm, re_hbm, ro_hbm, pos_out, neg_out,
           ua, va, ub, vb, uia, via, uib, vib, re_v, ro_v, sbuf,
           sua, sva, sub_, svb):
    c = lax.axis_index("c")
    s = lax.axis_index("s")

    pltpu.sync_copy(re_hbm, re_v)
    pltpu.sync_copy(ro_hbm, ro_v)
    re_regs = [re_v[pl.ds(j * L, L)] for j in range(DH // L)]
    ro_regs = [ro_v[pl.ds(j * L, L)] for j in range(DH // L)]
    lane = lax.iota(jnp.int32, L)
    onehots = [jnp.where(lane == k, 1.0, 0.0) for k in range(L)]

    def do_set(ei_hbm, out_hbm):
        def fire(b, ui, vi, ur, vr, su, sv):
            base = s * EPT + b * EB
            pltpu.sync_copy(ei_hbm.at[pl.ds(base, EB)], ui)
            pltpu.sync_copy(ei_hbm.at[pl.ds(E + base, EB)], vi)
            cu = pltpu.async_copy(h_hbm.at[ui], ur, su)
            cv = pltpu.async_copy(h_hbm.at[vi], vr, sv)
            return cu, cv

        def compute(b, ur, vr):
            def grp(q, _):
                vec = jnp.zeros((L,), jnp.float32)
                for k in range(L):
                    e = q * L + k
                    acc = None
                    for j in range(DH // L):
                        sl = pl.ds(j * L, L)
                        uw = ur[e, sl]
                        vw = vr[e, sl]
                        ulo = jax.lax.bitcast_convert_type(
                            jax.lax.shift_left(uw, 16), jnp.float32)
                        uhi = jax.lax.bitcast_convert_type(uw, jnp.float32)
                        vlo = jax.lax.bitcast_convert_type(
                            jax.lax.shift_left(vw, 16), jnp.float32)
                        vhi = jax.lax.bitcast_convert_type(vw, jnp.float32)
                        t = ulo * vlo * re_regs[j] + uhi * vhi * ro_regs[j]
                        acc = t if acc is None else acc + t
                    lanes = [acc[i] for i in range(L)]
                    while len(lanes) > 1:
                        lanes = [lanes[i] + lanes[i + 1]
                                 for i in range(0, len(lanes), 2)]
                    vec = vec + lanes[0] * onehots[k]
                sbuf[pl.ds(b * EB + q * L, L)] = vec
                return 0

            lax.fori_loop(0, EB // L, grp, 0)

        ca = fire(0, uia, via, ua, va, sua, sva)

        def wait(ur, su):
            pltpu.make_async_copy(h_hbm.at[uia], ur, su).wait()

        def pair(i, _):
            b = 2 * i
            wait(ua, sua)
            wait(va, sva)
            fire(b + 1, uib, vib, ub, vb, sub_, svb)
            compute(b, ua, va)
            wait(ub, sub_)
            wait(vb, svb)
            fire(b + 2, uia, via, ua, va, sua, sva)
            compute(b + 1, ub, vb)
            return 0

        lax.fori_loop(0, (NBLK - 1) // 2, pair, 0)
        wait(ua, sua)
        wait(va, sva)
        compute(NBLK - 1, ua, va)
        pltpu.sync_copy(sbuf, out_hbm.at[pl.ds(s * EPT, EPT)])

    @pl.when(c == 0)
    def _():
        do_set(pos_hbm, pos_out)

    @pl.when(c == 1)
    def _():
        do_set(neg_hbm, neg_out)


def kernel(x, block_edge_index, pos_edge_index, neg_edge_index, W_self, W_neigh, r):
    x32 = jax.lax.bitcast_convert_type(
        x.astype(jnp.bfloat16).reshape(N, D // 2, 2), jnp.int32)
    agg, degw = _seg_sum(x32, block_edge_index.reshape(-1))
    Wn_perm = jnp.concatenate([W_neigh[0::2], W_neigh[1::2]], axis=0)
    h = _emb(x, agg[:N], degw[:N].reshape(N, 1), W_self, Wn_perm)
    h32 = jax.lax.bitcast_convert_type(h.reshape(N, DH, 2), jnp.int32)
    pos_score, neg_score = _score(h32, pos_edge_index.reshape(-1),
                                  neg_edge_index.reshape(-1),
                                  r[0::2], r[1::2])
    return (pos_score, neg_score)
